# baseline (speedup 1.0000x reference)
import jax
import jax.numpy as jnp
from jax.experimental import pallas as pl
from jax.experimental.pallas import tpu as pltpu

E = 8
D = 1024
F = 128
N = 8192
BN = 1024
WCOLS = E * F + 128

_EPS = (-0.15443718433380127, 0.08470727503299713, -0.135980486869812,
        -0.15503625571727753, 1.266667366027832, 0.14829757809638977,
        2.1415603160858154, 1.0026742219924927)


def _fused_kernel(x_ref, wg_ref, wn_ref, wgb_ref, wnb_ref, eps_ref, w_ref,
                  b_ref, out_ref, wt_ref):
    @pl.when(pl.program_id(0) == 0)
    def _repack():
        for e in range(E):
            wt_ref[:, e * F:(e + 1) * F] = w_ref[e]
        pad = jnp.zeros((D, F - 2 * E), dtype=jnp.float32)
        wt_ref[:, E * F:E * F + E] = wg_ref[...]
        wt_ref[:, E * F + E:E * F + 2 * E] = wn_ref[...]
        wt_ref[:, E * F + 2 * E:] = pad

    x = x_ref[...]
    mm = jnp.dot(x, wt_ref[...], preferred_element_type=jnp.float32)
    g = mm[:, E * F:E * F + E] + wgb_ref[...]
    sp_in = mm[:, E * F + E:E * F + 2 * E] + wnb_ref[...]
    sp = jnp.maximum(sp_in, 0.0) + jnp.log1p(jnp.exp(-jnp.abs(sp_in)))
    h = g + eps_ref[...] * sp
    k = jnp.argmax(h, axis=1).reshape(BN, 1)
    for e in range(E):
        sel = (k >= e).astype(jnp.float32)
        chunk = (mm[:, e * F:(e + 1) * F] + b_ref[e].reshape(1, F)) * sel
        out_ref[:, e * F:(e + 1) * F] = chunk


def kernel(x, Wg_w, Wg_b, Wn_w, Wn_b, expert_w, expert_b):
    grid = (N // BN,)
    return pl.pallas_call(
        _fused_kernel,
        grid=grid,
        in_specs=[
            pl.BlockSpec((BN, D), lambda i: (i, 0)),
            pl.BlockSpec((D, E), lambda i: (0, 0)),
            pl.BlockSpec((D, E), lambda i: (0, 0)),
            pl.BlockSpec((1, E), lambda i: (0, 0)),
            pl.BlockSpec((1, E), lambda i: (0, 0)),
            pl.BlockSpec((1, E), lambda i: (0, 0)),
            pl.BlockSpec((E, D, F), lambda i: (0, 0, 0)),
            pl.BlockSpec((E, F), lambda i: (0, 0)),
        ],
        out_specs=pl.BlockSpec((BN, E * F), lambda i: (i, 0)),
        out_shape=jax.ShapeDtypeStruct((N, E * F), jnp.float32),
        scratch_shapes=[pltpu.VMEM((D, WCOLS), jnp.float32)],
    )(x, Wg_w, Wn_w, Wg_b.reshape(1, E), Wn_b.reshape(1, E),
      jnp.array([_EPS], dtype=jnp.float32), expert_w, expert_b)
